# baseline (speedup 1.0000x reference)
import jax
import jax.numpy as jnp
from jax import lax
from jax.experimental import pallas as pl
from jax.experimental.pallas import tpu as pltpu
from jax.experimental.pallas import tpu_sc as plsc

_N = 10000
_E = 320000
_D = 128
_NC = 2
_NS = 16
_NW = _NC * _NS
_CHUNK = 128
_NCHUNK = 80
_NA = _N + 3 * _NS
_ND = _N + 240
_WB = 632
_WBL = _N - (_NS - 1) * _WB
_SHIFT = 14
_MASK = (1 << _SHIFT) - 1

_mesh = plsc.VectorSubcoreMesh(core_axis_name="c", subcore_axis_name="s")


def _unpack_dst(packed_v, j, dbuf):
    row = packed_v.at[j]
    for k in range(_CHUNK // 16):
        p = row[pl.ds(k * 16, 16)]
        dbuf[pl.ds(k * 16, 16)] = lax.shift_right_logical(p, _SHIFT)


def _unpack_both(packed_v, j, sbuf, dbuf):
    row = packed_v.at[j]
    for k in range(_CHUNK // 16):
        p = row[pl.ds(k * 16, 16)]
        sbuf[pl.ds(k * 16, 16)] = p & _MASK
        dbuf[pl.ds(k * 16, 16)] = lax.shift_right_logical(p, _SHIFT)


def _deg_body(packed_hbm, ones_hbm, init_hbm, out_hbm,
              packed_v, ones_v, dbuf, acc):
    cid = lax.axis_index("c")
    sid = lax.axis_index("s")
    wid = cid * _NS + sid

    @pl.when(sid == 0)
    def _():
        pltpu.sync_copy(init_hbm.at[pl.ds(cid * _ND, _ND)], acc)

    pltpu.sync_copy(packed_hbm.at[wid], packed_v)
    pltpu.sync_copy(ones_hbm, ones_v)
    plsc.subcore_barrier()

    def chunk(j, carry):
        _unpack_dst(packed_v, j, dbuf)
        pltpu.sync_copy(ones_v, acc.at[dbuf], add=True)
        return carry

    lax.fori_loop(0, _NCHUNK, chunk, 0)
    plsc.subcore_barrier()

    @pl.when(sid == 0)
    def _():
        pltpu.sync_copy(acc, out_hbm.at[cid, 0])


_deg_kernel = pl.kernel(
    _deg_body,
    out_type=jax.ShapeDtypeStruct((_NC, 1, _ND), jnp.float32),
    mesh=_mesh,
    scratch_types=[
        pltpu.VMEM((_NCHUNK, _CHUNK), jnp.int32),
        pltpu.VMEM((_CHUNK,), jnp.float32),
        pltpu.VMEM((_CHUNK,), jnp.int32),
        pltpu.VMEM_SHARED((_ND,), jnp.float32),
    ],
)


def _agg_body(table_hbm, packed_hbm, zeros_hbm, out_hbm,
              packed_v, sbuf0, dbuf0, sbuf1, dbuf1, rows0, rows1,
              sem0, sem1, ssem0, ssem1, acc):
    cid = lax.axis_index("c")
    sid = lax.axis_index("s")
    wid = cid * _NS + sid

    @pl.when(jnp.logical_and(sid == 0, cid == 0))
    def _():
        pltpu.sync_copy(table_hbm, acc.at[pl.ds(0, _N)])

    @pl.when(jnp.logical_and(sid == 0, cid == 1))
    def _():
        pltpu.sync_copy(zeros_hbm, acc.at[pl.ds(0, _N)])

    pltpu.sync_copy(packed_hbm.at[wid], packed_v)
    plsc.subcore_barrier()

    _unpack_both(packed_v, 0, sbuf0, dbuf0)
    pltpu.async_copy(table_hbm.at[sbuf0], rows0, sem0)
    _unpack_both(packed_v, 1, sbuf1, dbuf1)
    pltpu.async_copy(table_hbm.at[sbuf1], rows1, sem1)

    def chunk(i, carry):
        j = 2 * i
        pltpu.make_async_copy(table_hbm.at[sbuf0], rows0, sem0).wait()
        pltpu.async_copy(rows0, acc.at[dbuf0], ssem0, add=True)
        pltpu.make_async_copy(table_hbm.at[sbuf1], rows1, sem1).wait()
        pltpu.async_copy(rows1, acc.at[dbuf1], ssem1, add=True)
        pltpu.make_async_copy(rows0, acc.at[dbuf0], ssem0).wait()
        _unpack_both(packed_v, j + 2, sbuf0, dbuf0)
        pltpu.async_copy(table_hbm.at[sbuf0], rows0, sem0)
        pltpu.make_async_copy(rows1, acc.at[dbuf1], ssem1).wait()
        _unpack_both(packed_v, j + 3, sbuf1, dbuf1)
        pltpu.async_copy(table_hbm.at[sbuf1], rows1, sem1)
        return carry

    lax.fori_loop(0, _NCHUNK // 2 - 1, chunk, 0)
    pltpu.make_async_copy(table_hbm.at[sbuf0], rows0, sem0).wait()
    pltpu.sync_copy(rows0, acc.at[dbuf0], add=True)
    pltpu.make_async_copy(table_hbm.at[sbuf1], rows1, sem1).wait()
    pltpu.sync_copy(rows1, acc.at[dbuf1], add=True)
    plsc.subcore_barrier()

    @pl.when(sid < _NS - 1)
    def _():
        pltpu.sync_copy(acc.at[pl.ds(sid * _WB, _WB)],
                        out_hbm.at[cid, pl.ds(sid * _WB, _WB)])

    @pl.when(sid == _NS - 1)
    def _():
        pltpu.sync_copy(acc.at[pl.ds((_NS - 1) * _WB, _WBL)],
                        out_hbm.at[cid, pl.ds((_NS - 1) * _WB, _WBL)])


_agg_kernel = pl.kernel(
    _agg_body,
    out_type=jax.ShapeDtypeStruct((_NC, _N, _D), jnp.float32),
    mesh=_mesh,
    scratch_types=[
        pltpu.VMEM((_NCHUNK, _CHUNK), jnp.int32),
        pltpu.VMEM((_CHUNK,), jnp.int32),
        pltpu.VMEM((_CHUNK,), jnp.int32),
        pltpu.VMEM((_CHUNK,), jnp.int32),
        pltpu.VMEM((_CHUNK,), jnp.int32),
        pltpu.VMEM((_CHUNK, _D), jnp.float32),
        pltpu.VMEM((_CHUNK, _D), jnp.float32),
        pltpu.SemaphoreType.DMA,
        pltpu.SemaphoreType.DMA,
        pltpu.SemaphoreType.DMA,
        pltpu.SemaphoreType.DMA,
        pltpu.VMEM_SHARED((_NA, _D), jnp.float32),
    ],
)


_BLK = 1000


def _prep_body(d0_ref, d1_ref, x_ref, xs_ref, dinv_ref):
    deg = d0_ref[...] + d1_ref[...]
    dinv = lax.rsqrt(deg)
    dinv_ref[...] = dinv
    xs_ref[...] = x_ref[...] * dinv


def _mm_body(p0_ref, p1_ref, dinv_ref, w1_ref, b1_ref, w2_ref, out_ref):
    t = p0_ref[...] + p1_ref[...]
    dinv = dinv_ref[...]
    a = jnp.dot(t, w1_ref[...], preferred_element_type=jnp.float32)
    h = jnp.maximum(a * dinv + b1_ref[...], 0.0)
    g = jnp.dot(h, w2_ref[...], preferred_element_type=jnp.float32)
    out_ref[...] = g * dinv


def _fin_body(q0_ref, q1_ref, dinv_ref, b2_ref, out_ref):
    out_ref[...] = (q0_ref[...] + q1_ref[...]) * dinv_ref[...] + b2_ref[...]


def _row_spec(cols):
    return pl.BlockSpec((_BLK, cols), lambda i: (i, 0))


def _full_spec(r, c):
    return pl.BlockSpec((r, c), lambda i: (0, 0))


_prep_call = pl.pallas_call(
    _prep_body,
    grid=(_N // _BLK,),
    in_specs=[_row_spec(1), _row_spec(1), _row_spec(_D)],
    out_specs=[_row_spec(_D), _row_spec(1)],
    out_shape=[
        jax.ShapeDtypeStruct((_N, _D), jnp.float32),
        jax.ShapeDtypeStruct((_N, 1), jnp.float32),
    ],
)

_mm_call = pl.pallas_call(
    _mm_body,
    grid=(_N // _BLK,),
    in_specs=[
        _row_spec(_D), _row_spec(_D), _row_spec(1),
        _full_spec(128, 256), _full_spec(1, 256), _full_spec(256, 128),
    ],
    out_specs=_row_spec(_D),
    out_shape=jax.ShapeDtypeStruct((_N, _D), jnp.float32),
)

_fin_call = pl.pallas_call(
    _fin_body,
    grid=(_N // _BLK,),
    in_specs=[_row_spec(_D), _row_spec(_D), _row_spec(1), _full_spec(1, _D)],
    out_specs=_row_spec(_D),
    out_shape=jax.ShapeDtypeStruct((_N, _D), jnp.float32),
)


def kernel(x, edge_index, W1, b1, W2, b2):
    ei = edge_index.astype(jnp.int32)
    packed = (ei[1] << _SHIFT) | ei[0]
    k3 = jnp.arange(3, dtype=jnp.int32)
    dummy = ((_N + 3 * (jnp.arange(_NW, dtype=jnp.int32) % _NS))[:, None]
             + k3[None, :]) << _SHIFT
    pad = jnp.broadcast_to(
        (dummy | k3[None, :])[:, None, :], (_NW, _NCHUNK, 3))
    packedp = jnp.concatenate(
        [packed.reshape(_NW, _NCHUNK, _CHUNK - 3), pad], axis=2)

    zeros_nd = jnp.zeros((_N, _D), jnp.float32)
    deg_init = jnp.concatenate(
        [jnp.ones((_ND,), jnp.float32), jnp.zeros((_ND,), jnp.float32)])
    ones_c = jnp.ones((_CHUNK,), jnp.float32)

    degp = _deg_kernel(packedp, ones_c, deg_init)
    d0 = degp[0, 0, :_N].reshape(_N, 1)
    d1 = degp[1, 0, :_N].reshape(_N, 1)
    xs, dinv = _prep_call(d0, d1, x)

    p = _agg_kernel(xs, packedp, zeros_nd)
    g2 = _mm_call(p[0], p[1], dinv, W1, b1.reshape(1, -1), W2)

    q = _agg_kernel(g2, packedp, zeros_nd)
    out = _fin_call(q[0], q[1], dinv, b2.reshape(1, -1))
    return out


# speedup vs baseline: 1.0011x; 1.0011x over previous
import jax
import jax.numpy as jnp
from jax import lax
from jax.experimental import pallas as pl
from jax.experimental.pallas import tpu as pltpu
from jax.experimental.pallas import tpu_sc as plsc

_N = 10000
_E = 320000
_D = 128
_NC = 2
_NS = 16
_NW = _NC * _NS
_CHUNK = 128
_NCHUNK = 80
_NA = _N + 3 * _NS
_ND = _N + 240
_WB = 632
_WBL = _N - (_NS - 1) * _WB
_SHIFT = 14
_MASK = (1 << _SHIFT) - 1

_mesh = plsc.VectorSubcoreMesh(core_axis_name="c", subcore_axis_name="s")


def _unpack_dst(packed_v, j, dbuf):
    row = packed_v.at[j]
    for k in range(_CHUNK // 16):
        p = row[pl.ds(k * 16, 16)]
        dbuf[pl.ds(k * 16, 16)] = lax.shift_right_logical(p, _SHIFT)


def _unpack_both(pbuf, sbuf, dbuf):
    for k in range(_CHUNK // 16):
        p = pbuf[pl.ds(k * 16, 16)]
        sbuf[pl.ds(k * 16, 16)] = p & _MASK
        dbuf[pl.ds(k * 16, 16)] = lax.shift_right_logical(p, _SHIFT)


def _deg_body(packed_hbm, ones_hbm, init_hbm, out_hbm,
              packed_v, ones_v, dbuf, acc):
    cid = lax.axis_index("c")
    sid = lax.axis_index("s")
    wid = cid * _NS + sid

    @pl.when(sid == 0)
    def _():
        pltpu.sync_copy(init_hbm.at[pl.ds(cid * _ND, _ND)], acc)

    pltpu.sync_copy(packed_hbm.at[wid], packed_v)
    pltpu.sync_copy(ones_hbm, ones_v)
    plsc.subcore_barrier()

    def chunk(j, carry):
        _unpack_dst(packed_v, j, dbuf)
        pltpu.sync_copy(ones_v, acc.at[dbuf], add=True)
        return carry

    lax.fori_loop(0, _NCHUNK, chunk, 0)
    plsc.subcore_barrier()

    @pl.when(sid == 0)
    def _():
        pltpu.sync_copy(acc, out_hbm.at[cid, 0])


_deg_kernel = pl.kernel(
    _deg_body,
    out_type=jax.ShapeDtypeStruct((_NC, 1, _ND), jnp.float32),
    mesh=_mesh,
    scratch_types=[
        pltpu.VMEM((_NCHUNK, _CHUNK), jnp.int32),
        pltpu.VMEM((_CHUNK,), jnp.float32),
        pltpu.VMEM((_CHUNK,), jnp.int32),
        pltpu.VMEM_SHARED((_ND,), jnp.float32),
    ],
)


def _agg_body(table_hbm, packedf_hbm, zeros_hbm, out_hbm,
              pb0, pb1, pb2, sb0, sb1, sb2, db0, db1, db2,
              rows0, rows1, rows2, gs0, gs1, gs2, ps0, ps1, ps2, acc):
    cid = lax.axis_index("c")
    sid = lax.axis_index("s")
    wid = cid * _NS + sid
    base = wid * _NCHUNK

    @pl.when(jnp.logical_and(sid == 0, cid == 0))
    def _():
        pltpu.sync_copy(table_hbm, acc.at[pl.ds(0, _N)])

    @pl.when(jnp.logical_and(sid == 0, cid == 1))
    def _():
        pltpu.sync_copy(zeros_hbm, acc.at[pl.ds(0, _N)])

    plsc.subcore_barrier()

    slots = ((pb0, sb0, db0, rows0, gs0, ps0),
             (pb1, sb1, db1, rows1, gs1, ps1),
             (pb2, sb2, db2, rows2, gs2, ps2))

    def _prow(j):
        return packedf_hbm.at[pl.ds((base + j) * _CHUNK, _CHUNK)]

    for b, (pb, sb, db, rr, gs, ps) in enumerate(slots):
        pltpu.sync_copy(_prow(b), pb)
        _unpack_both(pb, sb, db)
        pltpu.async_copy(table_hbm.at[sb], rr, gs)
        pltpu.async_copy(_prow(b + 3), pb, ps)

    def body(i, carry):
        for b, (pb, sb, db, rr, gs, ps) in enumerate(slots):
            j = 3 * i + b

            @pl.when(j < _NCHUNK)
            def _():
                pltpu.make_async_copy(table_hbm.at[sb], rr, gs).wait()
                pltpu.sync_copy(rr, acc.at[db], add=True)

                @pl.when(j + 3 < _NCHUNK)
                def _():
                    pltpu.make_async_copy(_prow(j + 3), pb, ps).wait()
                    _unpack_both(pb, sb, db)
                    pltpu.async_copy(table_hbm.at[sb], rr, gs)

                    @pl.when(j + 6 < _NCHUNK)
                    def _():
                        pltpu.async_copy(_prow(j + 6), pb, ps)
        return carry

    lax.fori_loop(0, (_NCHUNK + 2) // 3, body, 0)
    plsc.subcore_barrier()

    @pl.when(sid < _NS - 1)
    def _():
        pltpu.sync_copy(acc.at[pl.ds(sid * _WB, _WB)],
                        out_hbm.at[cid, pl.ds(sid * _WB, _WB)])

    @pl.when(sid == _NS - 1)
    def _():
        pltpu.sync_copy(acc.at[pl.ds((_NS - 1) * _WB, _WBL)],
                        out_hbm.at[cid, pl.ds((_NS - 1) * _WB, _WBL)])


_agg_kernel = pl.kernel(
    _agg_body,
    out_type=jax.ShapeDtypeStruct((_NC, _N, _D), jnp.float32),
    mesh=_mesh,
    scratch_types=(
        [pltpu.VMEM((_CHUNK,), jnp.int32)] * 9
        + [pltpu.VMEM((_CHUNK, _D), jnp.float32)] * 3
        + [pltpu.SemaphoreType.DMA] * 6
        + [pltpu.VMEM_SHARED((_NA, _D), jnp.float32)]
    ),
)


_BLK = 1000


def _prep_body(d0_ref, d1_ref, x_ref, xs_ref, dinv_ref):
    deg = d0_ref[...] + d1_ref[...]
    dinv = lax.rsqrt(deg)
    dinv_ref[...] = dinv
    xs_ref[...] = x_ref[...] * dinv


def _mm_body(p0_ref, p1_ref, dinv_ref, w1_ref, b1_ref, w2_ref, out_ref):
    t = p0_ref[...] + p1_ref[...]
    dinv = dinv_ref[...]
    a = jnp.dot(t, w1_ref[...], preferred_element_type=jnp.float32)
    h = jnp.maximum(a * dinv + b1_ref[...], 0.0)
    g = jnp.dot(h, w2_ref[...], preferred_element_type=jnp.float32)
    out_ref[...] = g * dinv


def _fin_body(q0_ref, q1_ref, dinv_ref, b2_ref, out_ref):
    out_ref[...] = (q0_ref[...] + q1_ref[...]) * dinv_ref[...] + b2_ref[...]


def _row_spec(cols):
    return pl.BlockSpec((_BLK, cols), lambda i: (i, 0))


def _full_spec(r, c):
    return pl.BlockSpec((r, c), lambda i: (0, 0))


_prep_call = pl.pallas_call(
    _prep_body,
    grid=(_N // _BLK,),
    in_specs=[_row_spec(1), _row_spec(1), _row_spec(_D)],
    out_specs=[_row_spec(_D), _row_spec(1)],
    out_shape=[
        jax.ShapeDtypeStruct((_N, _D), jnp.float32),
        jax.ShapeDtypeStruct((_N, 1), jnp.float32),
    ],
)

_mm_call = pl.pallas_call(
    _mm_body,
    grid=(_N // _BLK,),
    in_specs=[
        _row_spec(_D), _row_spec(_D), _row_spec(1),
        _full_spec(128, 256), _full_spec(1, 256), _full_spec(256, 128),
    ],
    out_specs=_row_spec(_D),
    out_shape=jax.ShapeDtypeStruct((_N, _D), jnp.float32),
)

_fin_call = pl.pallas_call(
    _fin_body,
    grid=(_N // _BLK,),
    in_specs=[_row_spec(_D), _row_spec(_D), _row_spec(1), _full_spec(1, _D)],
    out_specs=_row_spec(_D),
    out_shape=jax.ShapeDtypeStruct((_N, _D), jnp.float32),
)


def kernel(x, edge_index, W1, b1, W2, b2):
    ei = edge_index.astype(jnp.int32)
    packed = (ei[1] << _SHIFT) | ei[0]
    k3 = jnp.arange(3, dtype=jnp.int32)
    dummy = ((_N + 3 * (jnp.arange(_NW, dtype=jnp.int32) % _NS))[:, None]
             + k3[None, :]) << _SHIFT
    pad = jnp.broadcast_to(
        (dummy | k3[None, :])[:, None, :], (_NW, _NCHUNK, 3))
    packedp = jnp.concatenate(
        [packed.reshape(_NW, _NCHUNK, _CHUNK - 3), pad], axis=2)
    packedf = packedp.reshape(-1)

    zeros_nd = jnp.zeros((_N, _D), jnp.float32)
    deg_init = jnp.concatenate(
        [jnp.ones((_ND,), jnp.float32), jnp.zeros((_ND,), jnp.float32)])
    ones_c = jnp.ones((_CHUNK,), jnp.float32)

    degp = _deg_kernel(packedp, ones_c, deg_init)
    d0 = degp[0, 0, :_N].reshape(_N, 1)
    d1 = degp[1, 0, :_N].reshape(_N, 1)
    xs, dinv = _prep_call(d0, d1, x)

    p = _agg_kernel(xs, packedf, zeros_nd)
    g2 = _mm_call(p[0], p[1], dinv, W1, b1.reshape(1, -1), W2)

    q = _agg_kernel(g2, packedf, zeros_nd)
    out = _fin_call(q[0], q[1], dinv, b2.reshape(1, -1))
    return out


# speedup vs baseline: 1.2456x; 1.2443x over previous
import jax
import jax.numpy as jnp
from jax import lax
from jax.experimental import pallas as pl
from jax.experimental.pallas import tpu as pltpu
from jax.experimental.pallas import tpu_sc as plsc

_N = 10000
_E = 320000
_D = 128
_NC = 2
_NS = 16
_NW = _NC * _NS
_CHUNK = 128
_TOTCH = _E // _CHUNK
_NCHUNK = -(-_TOTCH // _NW)
_ND = 10112
_WB = 632
_WBL = _N - (_NS - 1) * _WB

_mesh = plsc.VectorSubcoreMesh(core_axis_name="c", subcore_axis_name="s")


def _deg_body(dst_hbm, ones_hbm, init_hbm, out_hbm,
              dbuf, ones_v, acc):
    cid = lax.axis_index("c")
    sid = lax.axis_index("s")
    wid = cid * _NS + sid
    base = wid * _NCHUNK

    @pl.when(sid == 0)
    def _():
        pltpu.sync_copy(init_hbm.at[pl.ds(cid * _ND, _ND)], acc)

    pltpu.sync_copy(ones_hbm, ones_v)
    plsc.subcore_barrier()

    def chunk(j, carry):
        @pl.when(base + j < _TOTCH)
        def _():
            pltpu.sync_copy(
                dst_hbm.at[pl.ds((base + j) * _CHUNK, _CHUNK)], dbuf)
            pltpu.sync_copy(ones_v, acc.at[dbuf], add=True)
        return carry

    lax.fori_loop(0, _NCHUNK, chunk, 0)
    plsc.subcore_barrier()

    @pl.when(sid == 0)
    def _():
        pltpu.sync_copy(acc, out_hbm.at[cid, 0])


_deg_kernel = pl.kernel(
    _deg_body,
    out_type=jax.ShapeDtypeStruct((_NC, 1, _ND), jnp.float32),
    mesh=_mesh,
    scratch_types=[
        pltpu.VMEM((_CHUNK,), jnp.int32),
        pltpu.VMEM((_CHUNK,), jnp.float32),
        pltpu.VMEM_SHARED((_ND,), jnp.float32),
    ],
)


def _agg_body(table_hbm, src_hbm, dst_hbm, zeros_hbm, out_hbm,
              sb0, sb1, sb2, db0, db1, db2, rows0, rows1, rows2,
              gs0, gs1, gs2, is0, is1, is2, acc):
    cid = lax.axis_index("c")
    sid = lax.axis_index("s")
    wid = cid * _NS + sid
    base = wid * _NCHUNK

    @pl.when(jnp.logical_and(sid == 0, cid == 0))
    def _():
        pltpu.sync_copy(table_hbm, acc)

    @pl.when(jnp.logical_and(sid == 0, cid == 1))
    def _():
        pltpu.sync_copy(zeros_hbm, acc)

    plsc.subcore_barrier()

    slots = ((sb0, db0, rows0, gs0, is0),
             (sb1, db1, rows1, gs1, is1),
             (sb2, db2, rows2, gs2, is2))

    def _idx(ref, j):
        return ref.at[pl.ds((base + j) * _CHUNK, _CHUNK)]

    for b, (sb, db, rr, gs, isem) in enumerate(slots):
        pltpu.sync_copy(_idx(src_hbm, b), sb)
        pltpu.sync_copy(_idx(dst_hbm, b), db)
        pltpu.async_copy(table_hbm.at[sb], rr, gs)

    def body(i, carry):
        for b, (sb, db, rr, gs, isem) in enumerate(slots):
            j = 3 * i + b

            @pl.when(jnp.logical_and(j < _NCHUNK, base + j < _TOTCH))
            def _():
                pltpu.make_async_copy(table_hbm.at[sb], rr, gs).wait()
                pltpu.sync_copy(rr, acc.at[db], add=True)

                @pl.when(jnp.logical_and(j + 3 < _NCHUNK,
                                         base + j + 3 < _TOTCH))
                def _():
                    pltpu.async_copy(_idx(src_hbm, j + 3), sb, isem)
                    pltpu.sync_copy(_idx(dst_hbm, j + 3), db)
                    pltpu.make_async_copy(
                        _idx(src_hbm, j + 3), sb, isem).wait()
                    pltpu.async_copy(table_hbm.at[sb], rr, gs)
        return carry

    lax.fori_loop(0, (_NCHUNK + 2) // 3, body, 0)
    plsc.subcore_barrier()

    @pl.when(sid < _NS - 1)
    def _():
        pltpu.sync_copy(acc.at[pl.ds(sid * _WB, _WB)],
                        out_hbm.at[cid, pl.ds(sid * _WB, _WB)])

    @pl.when(sid == _NS - 1)
    def _():
        pltpu.sync_copy(acc.at[pl.ds((_NS - 1) * _WB, _WBL)],
                        out_hbm.at[cid, pl.ds((_NS - 1) * _WB, _WBL)])


_agg_kernel = pl.kernel(
    _agg_body,
    out_type=jax.ShapeDtypeStruct((_NC, _N, _D), jnp.float32),
    mesh=_mesh,
    scratch_types=(
        [pltpu.VMEM((_CHUNK,), jnp.int32)] * 6
        + [pltpu.VMEM((_CHUNK, _D), jnp.float32)] * 3
        + [pltpu.SemaphoreType.DMA] * 6
        + [pltpu.VMEM_SHARED((_N, _D), jnp.float32)]
    ),
)


_BLK = 1000


def _prep_body(d0_ref, d1_ref, x_ref, xs_ref, dinv_ref):
    deg = d0_ref[...] + d1_ref[...]
    dinv = lax.rsqrt(deg)
    dinv_ref[...] = dinv
    xs_ref[...] = x_ref[...] * dinv


def _mm_body(p0_ref, p1_ref, dinv_ref, w1_ref, b1_ref, w2_ref, out_ref):
    t = p0_ref[...] + p1_ref[...]
    dinv = dinv_ref[...]
    a = jnp.dot(t, w1_ref[...], preferred_element_type=jnp.float32)
    h = jnp.maximum(a * dinv + b1_ref[...], 0.0)
    g = jnp.dot(h, w2_ref[...], preferred_element_type=jnp.float32)
    out_ref[...] = g * dinv


def _fin_body(q0_ref, q1_ref, dinv_ref, b2_ref, out_ref):
    out_ref[...] = (q0_ref[...] + q1_ref[...]) * dinv_ref[...] + b2_ref[...]


def _row_spec(cols):
    return pl.BlockSpec((_BLK, cols), lambda i: (i, 0))


def _full_spec(r, c):
    return pl.BlockSpec((r, c), lambda i: (0, 0))


_prep_call = pl.pallas_call(
    _prep_body,
    grid=(_N // _BLK,),
    in_specs=[_row_spec(1), _row_spec(1), _row_spec(_D)],
    out_specs=[_row_spec(_D), _row_spec(1)],
    out_shape=[
        jax.ShapeDtypeStruct((_N, _D), jnp.float32),
        jax.ShapeDtypeStruct((_N, 1), jnp.float32),
    ],
)

_mm_call = pl.pallas_call(
    _mm_body,
    grid=(_N // _BLK,),
    in_specs=[
        _row_spec(_D), _row_spec(_D), _row_spec(1),
        _full_spec(128, 256), _full_spec(1, 256), _full_spec(256, 128),
    ],
    out_specs=_row_spec(_D),
    out_shape=jax.ShapeDtypeStruct((_N, _D), jnp.float32),
)

_fin_call = pl.pallas_call(
    _fin_body,
    grid=(_N // _BLK,),
    in_specs=[_row_spec(_D), _row_spec(_D), _row_spec(1), _full_spec(1, _D)],
    out_specs=_row_spec(_D),
    out_shape=jax.ShapeDtypeStruct((_N, _D), jnp.float32),
)


def kernel(x, edge_index, W1, b1, W2, b2):
    ei = edge_index.astype(jnp.int32)
    src = ei[0]
    dst = ei[1]

    zeros_nd = jnp.zeros((_N, _D), jnp.float32)
    deg_init = jnp.concatenate(
        [jnp.ones((_ND,), jnp.float32), jnp.zeros((_ND,), jnp.float32)])
    ones_c = jnp.ones((_CHUNK,), jnp.float32)

    degp = _deg_kernel(dst, ones_c, deg_init)
    d0 = degp[0, 0, :_N].reshape(_N, 1)
    d1 = degp[1, 0, :_N].reshape(_N, 1)
    xs, dinv = _prep_call(d0, d1, x)

    p = _agg_kernel(xs, src, dst, zeros_nd)
    g2 = _mm_call(p[0], p[1], dinv, W1, b1.reshape(1, -1), W2)

    q = _agg_kernel(g2, src, dst, zeros_nd)
    out = _fin_call(q[0], q[1], dinv, b2.reshape(1, -1))
    return out
